# baseline (speedup 1.0000x reference)
import functools

import jax
import jax.numpy as jnp
from jax import lax
from jax.experimental import pallas as pl
from jax.experimental.pallas import tpu as pltpu
from jax.experimental.pallas import tpu_sc as plsc

N = 100000
D = 128
H = 64
M = 65536
K = 3
BN = 4096
NP = 102400
NT = NP // BN
NEG = float("-inf")


_PER_W = NP // 16


def _hist_body(hn_hbm, tn_hbm, out_hbm, idxh_v, idxt_v, ones_v, zbuf_v,
               hist_h_sh, hist_t_sh, sem):
  cid = lax.axis_index("c")
  sid = lax.axis_index("s")
  w = cid * 16 + sid

  for v in range(8):
    ones_v[pl.ds(v * 16, 16)] = jnp.ones((16,), jnp.float32)
  for v in range(128):
    zbuf_v[pl.ds(v * 16, 16)] = jnp.zeros((16,), jnp.float32)

  zb = sid * _PER_W
  zcopies = []
  for hist_sh in (hist_h_sh, hist_t_sh):
    for kchunk in range(3):
      zcopies.append(pltpu.async_copy(
          zbuf_v, hist_sh.at[pl.ds(zb + kchunk * 2048, 2048)], sem))
    zcopies.append(pltpu.async_copy(
        zbuf_v.at[pl.ds(0, 256)], hist_sh.at[pl.ds(zb + 3 * 2048, 256)], sem))
  pltpu.sync_copy(hn_hbm.at[pl.ds(w * 16, 16)], idxh_v)
  pltpu.sync_copy(tn_hbm.at[pl.ds(w * 16, 16)], idxt_v)
  for c in zcopies:
    c.wait()
  plsc.subcore_barrier()

  for idx_v, hist_sh in ((idxh_v, hist_h_sh), (idxt_v, hist_t_sh)):
    copies = [pltpu.async_copy(ones_v, hist_sh.at[idx_v.at[j]], sem, add=True)
              for j in range(16)]
    for c in copies:
      c.wait()
  plsc.subcore_barrier()

  c0 = pltpu.async_copy(hist_h_sh.at[pl.ds(zb, _PER_W)],
                        out_hbm.at[cid, 0, pl.ds(zb, _PER_W)], sem)
  c1 = pltpu.async_copy(hist_t_sh.at[pl.ds(zb, _PER_W)],
                        out_hbm.at[cid, 1, pl.ds(zb, _PER_W)], sem)
  c0.wait()
  c1.wait()


@functools.cache
def _hist_kernel_fn():
  return pl.kernel(
      _hist_body,
      out_type=jax.ShapeDtypeStruct((2, 2, NP), jnp.float32),
      mesh=plsc.VectorSubcoreMesh(core_axis_name="c", subcore_axis_name="s",
                                  num_cores=2, num_subcores=16),
      scratch_types=[
          pltpu.VMEM((16, 128), jnp.int32),
          pltpu.VMEM((16, 128), jnp.int32),
          pltpu.VMEM((128,), jnp.float32),
          pltpu.VMEM((2048,), jnp.float32),
          pltpu.VMEM_SHARED((NP,), jnp.float32),
          pltpu.VMEM_SHARED((NP,), jnp.float32),
          pltpu.SemaphoreType.DMA,
      ],
  )


def _hist_kernel(hn2, tn2):
  return _hist_kernel_fn()(hn2, tn2)


def _score_body(emb_ref, w1a_ref, w1b_ref, b1_ref, w2_ref, cen_ref, emb_hbm,
                sh_ref, st_ref, anc_v, cvec_v, sem):
  i = pl.program_id(0)

  @pl.when(i == 0)
  def _init():
    ch = pltpu.make_async_copy(emb_hbm.at[pl.ds(cen_ref[0], 1)],
                               anc_v.at[pl.ds(0, 1)], sem)
    ct = pltpu.make_async_copy(emb_hbm.at[pl.ds(cen_ref[1], 1)],
                               anc_v.at[pl.ds(1, 1)], sem)
    ch.start()
    ct.start()
    ch.wait()
    ct.wait()
    cvec_v[...] = (
        lax.dot_general(w1a_ref[...], anc_v[...], (((0,), (1,)), ((), ())),
                        preferred_element_type=jnp.float32)
        + b1_ref[...])

  xt = lax.dot_general(w1b_ref[...], emb_ref[...], (((0,), (1,)), ((), ())),
                       preferred_element_type=jnp.float32)
  hh = jnp.maximum(xt + cvec_v[:, 0:1], 0.0)
  ht = jnp.maximum(xt + cvec_v[:, 1:2], 0.0)
  sh = lax.dot_general(w2_ref[...], hh, (((0,), (0,)), ((), ())),
                       preferred_element_type=jnp.float32)
  st = lax.dot_general(w2_ref[...], ht, (((0,), (0,)), ((), ())),
                       preferred_element_type=jnp.float32)
  sh_ref[...] = sh.reshape(1, 1, BN)
  st_ref[...] = st.reshape(1, 1, BN)


def _scores(embed, W1a, W1b, b1, W2, centers):
  return pl.pallas_call(
      _score_body,
      grid=(NT,),
      in_specs=[
          pl.BlockSpec((BN, D), lambda i: (i, 0)),
          pl.BlockSpec((D, H), lambda i: (0, 0)),
          pl.BlockSpec((D, H), lambda i: (0, 0)),
          pl.BlockSpec((H, 1), lambda i: (0, 0)),
          pl.BlockSpec((H, 1), lambda i: (0, 0)),
          pl.BlockSpec(memory_space=pltpu.SMEM),
          pl.BlockSpec(memory_space=pltpu.HBM),
      ],
      out_specs=[
          pl.BlockSpec((1, 1, BN), lambda i: (i, 0, 0)),
          pl.BlockSpec((1, 1, BN), lambda i: (i, 0, 0)),
      ],
      out_shape=[
          jax.ShapeDtypeStruct((NT, 1, BN), jnp.float32),
          jax.ShapeDtypeStruct((NT, 1, BN), jnp.float32),
      ],
      scratch_shapes=[
          pltpu.VMEM((2, D), jnp.float32),
          pltpu.VMEM((H, 2), jnp.float32),
          pltpu.SemaphoreType.DMA,
      ],
  )(embed, W1a, W1b, b1, W2, centers, embed)


def _wsum_body(nf_ref, h3_ref, sh_ref, st_ref, ws_ref,
               cs_ref, cn_ref, cch_ref, cct_ref):
  i = pl.program_id(0)
  h3 = h3_ref[...]
  mh = h3[0][0:1, :] + h3[1][0:1, :]
  mt = h3[0][1:2, :] + h3[1][1:2, :]
  mm = jnp.concatenate([mh, mt], axis=0)
  rowid = i * BN + lax.broadcasted_iota(jnp.int32, (BN, 1), 0)
  blk = jnp.where(rowid < N, nf_ref[...], 0.0)
  contrib = lax.dot_general(mm, blk, (((1,), (0,)), ((), ())),
                            preferred_element_type=jnp.float32,
                            precision=lax.Precision.DEFAULT)

  @pl.when(i == 0)
  def _init():
    ws_ref[...] = jnp.zeros_like(ws_ref)

  ws_ref[...] += contrib

  lin = i * BN + lax.broadcasted_iota(jnp.int32, (1, BN), 1)
  lane = lax.broadcasted_iota(jnp.int32, (1, 1, 8), 2)
  big = jnp.int32(2 ** 30)
  rs = jnp.zeros((1, 1, 8), jnp.float32)
  rn = jnp.zeros((1, 1, 8), jnp.float32)
  rch = jnp.zeros((1, 1, 8), jnp.float32)
  rct = jnp.zeros((1, 1, 8), jnp.float32)
  for a, sc_ref in enumerate((sh_ref, st_ref)):
    sc = sc_ref[...][0]
    own = mh if a == 0 else mt
    s_w = jnp.where(own > 0.0, sc, NEG)
    for k in range(K):
      m = jnp.max(s_w)
      idx = jnp.min(jnp.where(s_w == m, lin, big))
      hit = lin == idx
      cnth = jnp.sum(jnp.where(hit, mh, 0.0))
      cntt = jnp.sum(jnp.where(hit, mt, 0.0))
      s_w = jnp.where(hit, NEG, s_w)
      sl = lane == (4 * a + k)
      rs = jnp.where(sl, m, rs)
      rn = jnp.where(sl, idx.astype(jnp.float32), rn)
      rch = jnp.where(sl, cnth, rch)
      rct = jnp.where(sl, cntt, rct)
  cs_ref[...] = rs
  cn_ref[...] = rn
  cch_ref[...] = rch
  cct_ref[...] = rct


def _wsum_cands(node_feat, hist, sh, st):
  cspec = pl.BlockSpec((1, 1, 8), lambda i: (i, 0, 0))
  cshape = jax.ShapeDtypeStruct((NT, 1, 8), jnp.float32)
  return pl.pallas_call(
      _wsum_body,
      grid=(NT,),
      in_specs=[
          pl.BlockSpec((BN, D), lambda i: (i, 0)),
          pl.BlockSpec((2, 2, BN), lambda i: (0, 0, i)),
          pl.BlockSpec((1, 1, BN), lambda i: (i, 0, 0)),
          pl.BlockSpec((1, 1, BN), lambda i: (i, 0, 0)),
      ],
      out_specs=[
          pl.BlockSpec((2, D), lambda i: (0, 0)),
          cspec, cspec, cspec, cspec,
      ],
      out_shape=[
          jax.ShapeDtypeStruct((2, D), jnp.float32),
          cshape, cshape, cshape, cshape,
      ],
  )(node_feat, hist, sh, st)
def _final_body(cs_ref, cn_ref, cch_ref, cct_ref, wsum_ref, wemb_ref, cen_ref,
                nf_hbm, out_ref, rows_v, sem):
  lane = lax.broadcasted_iota(jnp.int32, (NT, 1, 8), 2)
  bigf = jnp.float32(2.0 ** 30)
  c_s = cs_ref[...]
  c_n = cn_ref[...]
  c_ch = cch_ref[...]
  c_ct = cct_ref[...]

  copies = []
  sels = []
  uses = []
  cnts = [[], []]
  for a in range(2):
    amask = (lane >= 4 * a) & (lane < 4 * a + K)
    s_w = jnp.where(amask, c_s, NEG)
    rem = jnp.float32(K)
    for k in range(K):
      m = jnp.max(s_w)
      nsel = jnp.min(jnp.where(s_w == m, c_n, bigf))
      hit = (s_w == m) & (c_n == nsel)
      cnth = jnp.sum(jnp.where(hit, c_ch, 0.0))
      cntt = jnp.sum(jnp.where(hit, c_ct, 0.0))
      s_w = jnp.where(hit, NEG, s_w)
      idx = nsel.astype(jnp.int32)
      sels.append(idx)
      uses.append((rem > 0.0).astype(jnp.float32))
      cnts[0].append(cnth)
      cnts[1].append(cntt)
      rem = rem - (cnth if a == 0 else cntt)
      cp = pltpu.make_async_copy(nf_hbm.at[pl.ds(idx, 1)],
                                 rows_v.at[pl.ds(a * K + k, 1)], sem)
      cp.start()
      copies.append(cp)

  for a in range(2):
    cp = pltpu.make_async_copy(nf_hbm.at[pl.ds(cen_ref[a], 1)],
                               rows_v.at[pl.ds(2 * K + a, 1)], sem)
    cp.start()
    copies.append(cp)
  for cp in copies:
    cp.wait()

  eff = []
  for j in range(2 * K):
    d = uses[j]
    for i2 in range(j):
      d = d * (1.0 - uses[i2] * (sels[i2] == sels[j]).astype(jnp.float32))
    eff.append(d)

  vrows = []
  for a in range(2):
    corr = jnp.zeros((1, D), jnp.float32)
    for j in range(2 * K):
      corr = corr + (eff[j] * cnts[a][j]) * rows_v[j:j + 1, :]
    keep = jnp.float32(1.0)
    for j in range(2 * K):
      hit = uses[j] * (sels[j] == cen_ref[a]).astype(jnp.float32)
      keep = keep * (1.0 - hit)
    agg = (wsum_ref[a:a + 1, :] - corr) * jnp.float32(1.0 / M)
    vrows.append(keep * rows_v[2 * K + a:2 * K + a + 1, :] + agg)

  v = jnp.concatenate(vrows, axis=0)
  out_ref[...] = jnp.dot(v, wemb_ref[...], preferred_element_type=jnp.float32)


def _final(cs, cn, cch, cct, wsum, W_emb, centers, node_feat):
  cfull = pl.BlockSpec((NT, 1, 8), lambda: (0, 0, 0))
  return pl.pallas_call(
      _final_body,
      in_specs=[
          cfull, cfull, cfull, cfull,
          pl.BlockSpec((2, D), lambda: (0, 0)),
          pl.BlockSpec((D, D), lambda: (0, 0)),
          pl.BlockSpec(memory_space=pltpu.SMEM),
          pl.BlockSpec(memory_space=pltpu.HBM),
      ],
      out_specs=pl.BlockSpec((2, D), lambda: (0, 0)),
      out_shape=jax.ShapeDtypeStruct((2, D), jnp.float32),
      scratch_shapes=[
          pltpu.VMEM((2 * K + 2, D), jnp.float32),
          pltpu.SemaphoreType.DMA,
      ],
  )(cs, cn, cch, cct, wsum, W_emb, centers, node_feat)


def kernel(embed, node_feat, head_neighbors, tail_neighbors, head, tail,
           W1, b1, W2, b2, W_emb):
  centers = jnp.stack([head, tail]).astype(jnp.int32)
  hn2 = head_neighbors.reshape(M // 128, 128)
  tn2 = tail_neighbors.reshape(M // 128, 128)

  hist = _hist_kernel(hn2, tn2)

  W1a = W1[:D]
  W1b = W1[D:]
  sh, st = _scores(embed, W1a, W1b, b1.reshape(H, 1), W2, centers)
  wsum, cs, cn, cch, cct = _wsum_cands(node_feat, hist, sh, st)
  out = _final(cs, cn, cch, cct, wsum, W_emb, centers, node_feat)
  return out


# speedup vs baseline: 1.4586x; 1.4586x over previous
import functools

import jax
import jax.numpy as jnp
from jax import lax
from jax.experimental import pallas as pl
from jax.experimental.pallas import tpu as pltpu
from jax.experimental.pallas import tpu_sc as plsc

N = 100000
D = 128
H = 64
M = 65536
K = 3
BN = 4096
NP = 102400
NT = NP // BN
NR = NP // 128
NEG = float("-inf")

_PER_W = NP // 16


def _hist_body(hn_hbm, tn_hbm, out_hbm, idxh_v, idxt_v, ones_v, zbuf_v,
               hist_h_sh, hist_t_sh, sem):
  cid = lax.axis_index("c")
  sid = lax.axis_index("s")
  w = cid * 16 + sid

  for v in range(8):
    ones_v[pl.ds(v * 16, 16)] = jnp.ones((16,), jnp.float32)
  for v in range(128):
    zbuf_v[pl.ds(v * 16, 16)] = jnp.zeros((16,), jnp.float32)

  zb = sid * _PER_W
  zcopies = []
  for hist_sh in (hist_h_sh, hist_t_sh):
    for kchunk in range(3):
      zcopies.append(pltpu.async_copy(
          zbuf_v, hist_sh.at[pl.ds(zb + kchunk * 2048, 2048)], sem))
    zcopies.append(pltpu.async_copy(
        zbuf_v.at[pl.ds(0, 256)], hist_sh.at[pl.ds(zb + 3 * 2048, 256)], sem))
  pltpu.sync_copy(hn_hbm.at[pl.ds(w * 16, 16)], idxh_v)
  pltpu.sync_copy(tn_hbm.at[pl.ds(w * 16, 16)], idxt_v)
  for c in zcopies:
    c.wait()
  plsc.subcore_barrier()

  for idx_v, hist_sh in ((idxh_v, hist_h_sh), (idxt_v, hist_t_sh)):
    copies = [pltpu.async_copy(ones_v, hist_sh.at[idx_v.at[j]], sem, add=True)
              for j in range(16)]
    for c in copies:
      c.wait()
  plsc.subcore_barrier()

  c0 = pltpu.async_copy(hist_h_sh.at[pl.ds(zb, _PER_W)],
                        out_hbm.at[cid, 0, pl.ds(zb, _PER_W)], sem)
  c1 = pltpu.async_copy(hist_t_sh.at[pl.ds(zb, _PER_W)],
                        out_hbm.at[cid, 1, pl.ds(zb, _PER_W)], sem)
  c0.wait()
  c1.wait()


@functools.cache
def _hist_kernel_fn():
  return pl.kernel(
      _hist_body,
      out_type=jax.ShapeDtypeStruct((2, 2, NP), jnp.float32),
      mesh=plsc.VectorSubcoreMesh(core_axis_name="c", subcore_axis_name="s",
                                  num_cores=2, num_subcores=16),
      scratch_types=[
          pltpu.VMEM((16, 128), jnp.int32),
          pltpu.VMEM((16, 128), jnp.int32),
          pltpu.VMEM((128,), jnp.float32),
          pltpu.VMEM((2048,), jnp.float32),
          pltpu.VMEM_SHARED((NP,), jnp.float32),
          pltpu.VMEM_SHARED((NP,), jnp.float32),
          pltpu.SemaphoreType.DMA,
      ],
  )


def _hist_kernel(hn2, tn2):
  return _hist_kernel_fn()(hn2, tn2)


def _score_body(emb_ref, w1_ref, b1_ref, w2_ref, cen_ref, emb_hbm,
                sh_ref, st_ref, anc_v, cvec_v, sem):
  i = pl.program_id(0)

  @pl.when(i == 0)
  def _init():
    ch = pltpu.make_async_copy(emb_hbm.at[pl.ds(cen_ref[0], 1)],
                               anc_v.at[pl.ds(0, 1)], sem)
    ct = pltpu.make_async_copy(emb_hbm.at[pl.ds(cen_ref[1], 1)],
                               anc_v.at[pl.ds(1, 1)], sem)
    ch.start()
    ct.start()
    ch.wait()
    ct.wait()
    cvec_v[...] = (
        lax.dot_general(w1_ref[0:D, :], anc_v[...], (((0,), (1,)), ((), ())),
                        preferred_element_type=jnp.float32)
        + b1_ref[...])

  xt = lax.dot_general(w1_ref[D:2 * D, :], emb_ref[...],
                       (((0,), (1,)), ((), ())),
                       preferred_element_type=jnp.float32)
  hh = jnp.maximum(xt + cvec_v[:, 0:1], 0.0)
  ht = jnp.maximum(xt + cvec_v[:, 1:2], 0.0)
  sh = lax.dot_general(w2_ref[...], hh, (((0,), (0,)), ((), ())),
                       preferred_element_type=jnp.float32)
  st = lax.dot_general(w2_ref[...], ht, (((0,), (0,)), ((), ())),
                       preferred_element_type=jnp.float32)
  sh_ref[...] = sh.reshape(1, 1, BN)
  st_ref[...] = st.reshape(1, 1, BN)


def _scores(embed, W1, b1, W2, centers):
  return pl.pallas_call(
      _score_body,
      grid=(NT,),
      in_specs=[
          pl.BlockSpec((BN, D), lambda i: (i, 0)),
          pl.BlockSpec((2 * D, H), lambda i: (0, 0)),
          pl.BlockSpec((H, 1), lambda i: (0, 0)),
          pl.BlockSpec((H, 1), lambda i: (0, 0)),
          pl.BlockSpec(memory_space=pltpu.SMEM),
          pl.BlockSpec(memory_space=pltpu.HBM),
      ],
      out_specs=[
          pl.BlockSpec((1, 1, BN), lambda i: (i, 0, 0)),
          pl.BlockSpec((1, 1, BN), lambda i: (i, 0, 0)),
      ],
      out_shape=[
          jax.ShapeDtypeStruct((NT, 1, BN), jnp.float32),
          jax.ShapeDtypeStruct((NT, 1, BN), jnp.float32),
      ],
      scratch_shapes=[
          pltpu.VMEM((2, D), jnp.float32),
          pltpu.VMEM((H, 2), jnp.float32),
          pltpu.SemaphoreType.DMA,
      ],
  )(embed, W1, b1, W2, centers, embed)


def _wsum_body(nf_ref, h3_ref, ws_ref):
  i = pl.program_id(0)
  h3 = h3_ref[...]
  mh = h3[0][0:1, :] + h3[1][0:1, :]
  mt = h3[0][1:2, :] + h3[1][1:2, :]
  mm = jnp.concatenate([mh, mt], axis=0)
  rowid = i * BN + lax.broadcasted_iota(jnp.int32, (BN, 1), 0)
  blk = jnp.where(rowid < N, nf_ref[...], 0.0)
  contrib = lax.dot_general(mm, blk, (((1,), (0,)), ((), ())),
                            preferred_element_type=jnp.float32,
                            precision=lax.Precision.DEFAULT)

  @pl.when(i == 0)
  def _init():
    ws_ref[...] = jnp.zeros_like(ws_ref)

  ws_ref[...] += contrib


def _wsum(node_feat, hist):
  return pl.pallas_call(
      _wsum_body,
      grid=(NT,),
      in_specs=[
          pl.BlockSpec((BN, D), lambda i: (i, 0)),
          pl.BlockSpec((2, 2, BN), lambda i: (0, 0, i)),
      ],
      out_specs=pl.BlockSpec((2, D), lambda i: (0, 0)),
      out_shape=jax.ShapeDtypeStruct((2, D), jnp.float32),
  )(node_feat, hist)


def _final_body(sh_ref, st_ref, h4_ref, wsum_ref, wemb_ref, cen_ref, nf_hbm,
                out_ref, rows_v, sem):
  lin = (lax.broadcasted_iota(jnp.int32, (NR, 128), 0) * 128
         + lax.broadcasted_iota(jnp.int32, (NR, 128), 1))
  big = jnp.int32(2 ** 30)

  h4 = h4_ref[...]
  mult_h = h4[0, 0] + h4[1, 0]
  mult_t = h4[0, 1] + h4[1, 1]

  copies = []
  sels = []
  uses = []
  cnts = [[], []]
  for a, (sc_ref, own) in enumerate(
      ((sh_ref, mult_h), (st_ref, mult_t))):
    s_w = jnp.where(own > 0.0, sc_ref[...], NEG)
    rem = jnp.float32(K)
    for k in range(K):
      m = jnp.max(s_w)
      idx = jnp.min(jnp.where(s_w == m, lin, big))
      hit = lin == idx
      cnth = jnp.sum(jnp.where(hit, mult_h, 0.0))
      cntt = jnp.sum(jnp.where(hit, mult_t, 0.0))
      sels.append(idx)
      uses.append((rem > 0.0).astype(jnp.float32))
      cnts[0].append(cnth)
      cnts[1].append(cntt)
      rem = rem - (cnth if a == 0 else cntt)
      s_w = jnp.where(hit, NEG, s_w)
      cp = pltpu.make_async_copy(nf_hbm.at[pl.ds(idx, 1)],
                                 rows_v.at[pl.ds(a * K + k, 1)], sem)
      cp.start()
      copies.append(cp)

  for a in range(2):
    cp = pltpu.make_async_copy(nf_hbm.at[pl.ds(cen_ref[a], 1)],
                               rows_v.at[pl.ds(2 * K + a, 1)], sem)
    cp.start()
    copies.append(cp)
  for cp in copies:
    cp.wait()

  eff = []
  for j in range(2 * K):
    d = uses[j]
    for i2 in range(j):
      d = d * (1.0 - uses[i2] * (sels[i2] == sels[j]).astype(jnp.float32))
    eff.append(d)

  vrows = []
  for a in range(2):
    corr = jnp.zeros((1, D), jnp.float32)
    for j in range(2 * K):
      corr = corr + (eff[j] * cnts[a][j]) * rows_v[j:j + 1, :]
    keep = jnp.float32(1.0)
    for j in range(2 * K):
      hit = uses[j] * (sels[j] == cen_ref[a]).astype(jnp.float32)
      keep = keep * (1.0 - hit)
    agg = (wsum_ref[a:a + 1, :] - corr) * jnp.float32(1.0 / M)
    vrows.append(keep * rows_v[2 * K + a:2 * K + a + 1, :] + agg)

  v = jnp.concatenate(vrows, axis=0)
  out_ref[...] = jnp.dot(v, wemb_ref[...], preferred_element_type=jnp.float32)


def _final(sh, st, hist4, wsum, W_emb, centers, node_feat):
  full = pl.BlockSpec((NR, 128), lambda: (0, 0))
  return pl.pallas_call(
      _final_body,
      in_specs=[
          full, full,
          pl.BlockSpec((2, 2, NR, 128), lambda: (0, 0, 0, 0)),
          pl.BlockSpec((2, D), lambda: (0, 0)),
          pl.BlockSpec((D, D), lambda: (0, 0)),
          pl.BlockSpec(memory_space=pltpu.SMEM),
          pl.BlockSpec(memory_space=pltpu.HBM),
      ],
      out_specs=pl.BlockSpec((2, D), lambda: (0, 0)),
      out_shape=jax.ShapeDtypeStruct((2, D), jnp.float32),
      scratch_shapes=[
          pltpu.VMEM((2 * K + 2, D), jnp.float32),
          pltpu.SemaphoreType.DMA,
      ],
  )(sh, st, hist4, wsum, W_emb, centers, node_feat)


def kernel(embed, node_feat, head_neighbors, tail_neighbors, head, tail,
           W1, b1, W2, b2, W_emb):
  centers = jnp.stack([head, tail]).astype(jnp.int32)
  hn2 = head_neighbors.reshape(M // 128, 128)
  tn2 = tail_neighbors.reshape(M // 128, 128)

  hist = _hist_kernel(hn2, tn2)
  hist4 = hist.reshape(2, 2, NR, 128)

  sh, st = _scores(embed, W1, b1.reshape(H, 1), W2, centers)
  wsum = _wsum(node_feat, hist)

  out = _final(sh.reshape(NR, 128), st.reshape(NR, 128),
               hist4, wsum, W_emb, centers, node_feat)
  return out


# speedup vs baseline: 1.4792x; 1.0142x over previous
import functools

import jax
import jax.numpy as jnp
from jax import lax
from jax.experimental import pallas as pl
from jax.experimental.pallas import tpu as pltpu
from jax.experimental.pallas import tpu_sc as plsc

N = 100000
D = 128
H = 64
M = 65536
K = 3
BN = 4096
NP = 102400
NT = NP // BN
NR = NP // 128
NEG = float("-inf")

_PER_W = NP // 16


def _hist_body(hn_hbm, tn_hbm, out_hbm, idxh_v, idxt_v, ones_v, zbuf_v,
               hist_h_sh, hist_t_sh, sem):
  cid = lax.axis_index("c")
  sid = lax.axis_index("s")
  w = cid * 16 + sid

  for v in range(8):
    ones_v[pl.ds(v * 16, 16)] = jnp.ones((16,), jnp.float32)
  for v in range(128):
    zbuf_v[pl.ds(v * 16, 16)] = jnp.zeros((16,), jnp.float32)

  zb = sid * _PER_W
  zcopies = []
  for hist_sh in (hist_h_sh, hist_t_sh):
    for kchunk in range(3):
      zcopies.append(pltpu.async_copy(
          zbuf_v, hist_sh.at[pl.ds(zb + kchunk * 2048, 2048)], sem))
    zcopies.append(pltpu.async_copy(
        zbuf_v.at[pl.ds(0, 256)], hist_sh.at[pl.ds(zb + 3 * 2048, 256)], sem))
  pltpu.sync_copy(hn_hbm.at[pl.ds(w * 16, 16)], idxh_v)
  pltpu.sync_copy(tn_hbm.at[pl.ds(w * 16, 16)], idxt_v)
  for c in zcopies:
    c.wait()
  plsc.subcore_barrier()

  for idx_v, hist_sh in ((idxh_v, hist_h_sh), (idxt_v, hist_t_sh)):
    copies = [pltpu.async_copy(ones_v, hist_sh.at[idx_v.at[j]], sem, add=True)
              for j in range(16)]
    for c in copies:
      c.wait()
  plsc.subcore_barrier()

  c0 = pltpu.async_copy(hist_h_sh.at[pl.ds(zb, _PER_W)],
                        out_hbm.at[cid, 0, pl.ds(zb, _PER_W)], sem)
  c1 = pltpu.async_copy(hist_t_sh.at[pl.ds(zb, _PER_W)],
                        out_hbm.at[cid, 1, pl.ds(zb, _PER_W)], sem)
  c0.wait()
  c1.wait()


@functools.cache
def _hist_kernel_fn():
  return pl.kernel(
      _hist_body,
      out_type=jax.ShapeDtypeStruct((2, 2, NP), jnp.float32),
      mesh=plsc.VectorSubcoreMesh(core_axis_name="c", subcore_axis_name="s",
                                  num_cores=2, num_subcores=16),
      scratch_types=[
          pltpu.VMEM((16, 128), jnp.int32),
          pltpu.VMEM((16, 128), jnp.int32),
          pltpu.VMEM((128,), jnp.float32),
          pltpu.VMEM((2048,), jnp.float32),
          pltpu.VMEM_SHARED((NP,), jnp.float32),
          pltpu.VMEM_SHARED((NP,), jnp.float32),
          pltpu.SemaphoreType.DMA,
      ],
  )


def _hist_kernel(hn2, tn2):
  return _hist_kernel_fn()(hn2, tn2)


def _score_body(emb_ref, w1_ref, b1_ref, w2_ref, cen_ref, emb_hbm,
                sh_ref, st_ref, anc_v, cvec_v, sem):
  i = pl.program_id(0)

  @pl.when(i == 0)
  def _init():
    ch = pltpu.make_async_copy(emb_hbm.at[pl.ds(cen_ref[0], 1)],
                               anc_v.at[pl.ds(0, 1)], sem)
    ct = pltpu.make_async_copy(emb_hbm.at[pl.ds(cen_ref[1], 1)],
                               anc_v.at[pl.ds(1, 1)], sem)
    ch.start()
    ct.start()
    ch.wait()
    ct.wait()
    cvec_v[...] = (
        lax.dot_general(w1_ref[0:D, :], anc_v[...], (((0,), (1,)), ((), ())),
                        preferred_element_type=jnp.float32)
        + b1_ref[...])

  xt = lax.dot_general(w1_ref[D:2 * D, :], emb_ref[...],
                       (((0,), (1,)), ((), ())),
                       preferred_element_type=jnp.float32)
  hh = jnp.maximum(xt + cvec_v[:, 0:1], 0.0)
  ht = jnp.maximum(xt + cvec_v[:, 1:2], 0.0)
  w2c = w2_ref[...]
  sh = jnp.sum(hh * w2c, axis=0, keepdims=True)
  st = jnp.sum(ht * w2c, axis=0, keepdims=True)
  sh_ref[...] = sh.reshape(1, 1, BN)
  st_ref[...] = st.reshape(1, 1, BN)


def _scores(embed, W1, b1, W2, centers):
  return pl.pallas_call(
      _score_body,
      grid=(NT,),
      in_specs=[
          pl.BlockSpec((BN, D), lambda i: (i, 0)),
          pl.BlockSpec((2 * D, H), lambda i: (0, 0)),
          pl.BlockSpec((H, 1), lambda i: (0, 0)),
          pl.BlockSpec((H, 1), lambda i: (0, 0)),
          pl.BlockSpec(memory_space=pltpu.SMEM),
          pl.BlockSpec(memory_space=pltpu.HBM),
      ],
      out_specs=[
          pl.BlockSpec((1, 1, BN), lambda i: (i, 0, 0)),
          pl.BlockSpec((1, 1, BN), lambda i: (i, 0, 0)),
      ],
      out_shape=[
          jax.ShapeDtypeStruct((NT, 1, BN), jnp.float32),
          jax.ShapeDtypeStruct((NT, 1, BN), jnp.float32),
      ],
      scratch_shapes=[
          pltpu.VMEM((2, D), jnp.float32),
          pltpu.VMEM((H, 2), jnp.float32),
          pltpu.SemaphoreType.DMA,
      ],
  )(embed, W1, b1, W2, centers, embed)


def _wsum_body(nf_ref, h3_ref, ws_ref):
  i = pl.program_id(0)
  h3 = h3_ref[...]
  mh = h3[0][0:1, :] + h3[1][0:1, :]
  mt = h3[0][1:2, :] + h3[1][1:2, :]
  mm = jnp.concatenate([mh, mt], axis=0)
  rowid = i * BN + lax.broadcasted_iota(jnp.int32, (BN, 1), 0)
  blk = jnp.where(rowid < N, nf_ref[...], 0.0)
  contrib = lax.dot_general(mm, blk, (((1,), (0,)), ((), ())),
                            preferred_element_type=jnp.float32,
                            precision=lax.Precision.DEFAULT)

  @pl.when(i == 0)
  def _init():
    ws_ref[...] = jnp.zeros_like(ws_ref)

  ws_ref[...] += contrib


def _wsum(node_feat, hist):
  return pl.pallas_call(
      _wsum_body,
      grid=(NT,),
      in_specs=[
          pl.BlockSpec((BN, D), lambda i: (i, 0)),
          pl.BlockSpec((2, 2, BN), lambda i: (0, 0, i)),
      ],
      out_specs=pl.BlockSpec((2, D), lambda i: (0, 0)),
      out_shape=jax.ShapeDtypeStruct((2, D), jnp.float32),
  )(node_feat, hist)


def _final_body(sh_ref, st_ref, h4_ref, wsum_ref, wemb_ref, cen_ref, nf_hbm,
                out_ref, rows_v, sem):
  lin = (lax.broadcasted_iota(jnp.int32, (NR, 128), 0) * 128
         + lax.broadcasted_iota(jnp.int32, (NR, 128), 1))
  big = jnp.int32(2 ** 30)

  h4 = h4_ref[...]
  mult_h = h4[0, 0] + h4[1, 0]
  mult_t = h4[0, 1] + h4[1, 1]

  copies = []
  sels = []
  uses = []
  cnts = [[], []]
  for a, (sc_ref, own) in enumerate(
      ((sh_ref, mult_h), (st_ref, mult_t))):
    s_w = jnp.where(own > 0.0, sc_ref[...], NEG)
    rem = jnp.float32(K)
    for k in range(K):
      m = jnp.max(s_w)
      idx = jnp.min(jnp.where(s_w == m, lin, big))
      hit = lin == idx
      cnth = jnp.sum(jnp.where(hit, mult_h, 0.0))
      cntt = jnp.sum(jnp.where(hit, mult_t, 0.0))
      sels.append(idx)
      uses.append((rem > 0.0).astype(jnp.float32))
      cnts[0].append(cnth)
      cnts[1].append(cntt)
      rem = rem - (cnth if a == 0 else cntt)
      s_w = jnp.where(hit, NEG, s_w)
      cp = pltpu.make_async_copy(nf_hbm.at[pl.ds(idx, 1)],
                                 rows_v.at[pl.ds(a * K + k, 1)], sem)
      cp.start()
      copies.append(cp)

  for a in range(2):
    cp = pltpu.make_async_copy(nf_hbm.at[pl.ds(cen_ref[a], 1)],
                               rows_v.at[pl.ds(2 * K + a, 1)], sem)
    cp.start()
    copies.append(cp)
  for cp in copies:
    cp.wait()

  eff = []
  for j in range(2 * K):
    d = uses[j]
    for i2 in range(j):
      d = d * (1.0 - uses[i2] * (sels[i2] == sels[j]).astype(jnp.float32))
    eff.append(d)

  vrows = []
  for a in range(2):
    corr = jnp.zeros((1, D), jnp.float32)
    for j in range(2 * K):
      corr = corr + (eff[j] * cnts[a][j]) * rows_v[j:j + 1, :]
    keep = jnp.float32(1.0)
    for j in range(2 * K):
      hit = uses[j] * (sels[j] == cen_ref[a]).astype(jnp.float32)
      keep = keep * (1.0 - hit)
    agg = (wsum_ref[a:a + 1, :] - corr) * jnp.float32(1.0 / M)
    vrows.append(keep * rows_v[2 * K + a:2 * K + a + 1, :] + agg)

  v = jnp.concatenate(vrows, axis=0)
  out_ref[...] = jnp.dot(v, wemb_ref[...], preferred_element_type=jnp.float32)


def _final(sh, st, hist4, wsum, W_emb, centers, node_feat):
  full = pl.BlockSpec((NR, 128), lambda: (0, 0))
  return pl.pallas_call(
      _final_body,
      in_specs=[
          full, full,
          pl.BlockSpec((2, 2, NR, 128), lambda: (0, 0, 0, 0)),
          pl.BlockSpec((2, D), lambda: (0, 0)),
          pl.BlockSpec((D, D), lambda: (0, 0)),
          pl.BlockSpec(memory_space=pltpu.SMEM),
          pl.BlockSpec(memory_space=pltpu.HBM),
      ],
      out_specs=pl.BlockSpec((2, D), lambda: (0, 0)),
      out_shape=jax.ShapeDtypeStruct((2, D), jnp.float32),
      scratch_shapes=[
          pltpu.VMEM((2 * K + 2, D), jnp.float32),
          pltpu.SemaphoreType.DMA,
      ],
  )(sh, st, hist4, wsum, W_emb, centers, node_feat)


def kernel(embed, node_feat, head_neighbors, tail_neighbors, head, tail,
           W1, b1, W2, b2, W_emb):
  centers = jnp.stack([head, tail]).astype(jnp.int32)
  hn2 = head_neighbors.reshape(M // 128, 128)
  tn2 = tail_neighbors.reshape(M // 128, 128)

  hist = _hist_kernel(hn2, tn2)
  hist4 = hist.reshape(2, 2, NR, 128)

  sh, st = _scores(embed, W1, b1.reshape(H, 1), W2, centers)
  wsum = _wsum(node_feat, hist)

  out = _final(sh.reshape(NR, 128), st.reshape(NR, 128),
               hist4, wsum, W_emb, centers, node_feat)
  return out
